# Initial kernel scaffold; baseline (speedup 1.0000x reference)
import jax
import jax.numpy as jnp
from jax.experimental import pallas as pl


def kernel(x, edge_index, batch, depth, W1, b1, W2, b2, W_out, b_out):
    raise NotImplementedError("write your pallas kernel here")


# speedup vs baseline: 15.8962x; 15.8962x over previous
import functools

import jax
import jax.numpy as jnp
from jax import lax
from jax.experimental import pallas as pl
from jax.experimental.pallas import tpu as pltpu
from jax.experimental.pallas import tpu_sc as plsc

N = 10000
NN = 10240
E = 320000
H = 64
D_IN = 128
G = 64
NTILES = 32
NSUB = 16
CK = 128
NCK = 80
EPAD = NTILES * NCK * CK
RPT = NN // NSUB
DEGW = 16
RBLK = 2560


def _deg_body(dst_hbm, ones_hbm, zero_hbm, out_hbm, dst_v, ones_v, acc_sh, sem):
    c = lax.axis_index("c")
    s = lax.axis_index("s")
    wid = c * NSUB + s
    pltpu.sync_copy(zero_hbm, acc_sh.at[pl.ds(s * RPT, RPT)])
    pltpu.sync_copy(dst_hbm.at[wid], dst_v)
    pltpu.sync_copy(ones_hbm, ones_v)
    plsc.subcore_barrier()

    def body(j, carry):
        pltpu.sync_copy(ones_v, acc_sh.at[dst_v.at[j]], add=True)
        return carry

    lax.fori_loop(0, NCK, body, 0)
    plsc.subcore_barrier()
    pltpu.sync_copy(acc_sh.at[pl.ds(s * RPT, RPT)],
                    out_hbm.at[c, pl.ds(s * RPT, RPT)])
    del sem


_deg_call = functools.partial(
    pl.kernel,
    out_type=jax.ShapeDtypeStruct((2, NN, DEGW), jnp.float32),
    mesh=plsc.VectorSubcoreMesh(core_axis_name="c", subcore_axis_name="s"),
    compiler_params=pltpu.CompilerParams(use_tc_tiling_on_sc=False),
    scratch_types=[
        pltpu.VMEM((NCK, CK), jnp.int32),
        pltpu.VMEM((CK, DEGW), jnp.float32),
        pltpu.VMEM_SHARED((NN, DEGW), jnp.float32),
        pltpu.SemaphoreType.DMA,
    ],
)(_deg_body)


def _mp_body(src_hbm, dst_hbm, hs_hbm, zero_hbm, out_hbm,
             src_v, dst_v, ra, rb, acc_sh, sa, sb):
    c = lax.axis_index("c")
    s = lax.axis_index("s")
    wid = c * NSUB + s
    pltpu.sync_copy(zero_hbm, acc_sh.at[pl.ds(s * RPT, RPT)])
    pltpu.sync_copy(src_hbm.at[wid], src_v)
    pltpu.sync_copy(dst_hbm.at[wid], dst_v)
    plsc.subcore_barrier()

    pltpu.async_copy(hs_hbm.at[src_v.at[0]], ra, sa)

    def pair(i, carry):
        j = 2 * i
        pltpu.async_copy(hs_hbm.at[src_v.at[j + 1]], rb, sb)
        pltpu.make_async_copy(hs_hbm.at[src_v.at[j]], ra, sa).wait()
        pltpu.sync_copy(ra, acc_sh.at[dst_v.at[j]], add=True)

        @pl.when(j + 2 < NCK)
        def _():
            pltpu.async_copy(hs_hbm.at[src_v.at[j + 2]], ra, sa)

        pltpu.make_async_copy(hs_hbm.at[src_v.at[j + 1]], rb, sb).wait()
        pltpu.sync_copy(rb, acc_sh.at[dst_v.at[j + 1]], add=True)
        return carry

    lax.fori_loop(0, NCK // 2, pair, 0)
    plsc.subcore_barrier()
    pltpu.sync_copy(acc_sh.at[pl.ds(s * RPT, RPT)],
                    out_hbm.at[c, pl.ds(s * RPT, RPT)])


_mp_call = functools.partial(
    pl.kernel,
    out_type=jax.ShapeDtypeStruct((2, NN, H), jnp.float32),
    mesh=plsc.VectorSubcoreMesh(core_axis_name="c", subcore_axis_name="s"),
    compiler_params=pltpu.CompilerParams(use_tc_tiling_on_sc=False),
    scratch_types=[
        pltpu.VMEM((NCK, CK), jnp.int32),
        pltpu.VMEM((NCK, CK), jnp.int32),
        pltpu.VMEM((CK, H), jnp.float32),
        pltpu.VMEM((CK, H), jnp.float32),
        pltpu.VMEM_SHARED((NN, H), jnp.float32),
        pltpu.SemaphoreType.DMA,
        pltpu.SemaphoreType.DMA,
    ],
)(_mp_body)


def _dinv_from(d0_ref, d1_ref):
    deg = d0_ref[:, :1] + d1_ref[:, :1] + 1.0
    return lax.rsqrt(deg)


def _tca_body(x_ref, w1_ref, d0_ref, d1_ref, hs_ref):
    dinv = _dinv_from(d0_ref, d1_ref)
    h = jnp.dot(x_ref[...], w1_ref[...], preferred_element_type=jnp.float32)
    hs_ref[...] = h * dinv


def _tcd_body(a0_ref, a1_ref, hs1_ref, d0_ref, d1_ref, b1_ref, w2_ref, hs2_ref):
    dinv = _dinv_from(d0_ref, d1_ref)
    agg = dinv * (a0_ref[...] + a1_ref[...] + hs1_ref[...]) + b1_ref[...]
    out1 = jnp.maximum(agg, 0.0)
    h2 = jnp.dot(out1, w2_ref[...], preferred_element_type=jnp.float32)
    hs2_ref[...] = h2 * dinv


def _tcf_body(a0_ref, a1_ref, hs2_ref, d0_ref, d1_ref, b2_ref, batch_ref,
              depth_ref, wpool_ref, wd_ref, bout_ref, out_ref,
              sums_scr, cnts_scr):
    i = pl.program_id(0)

    @pl.when(i == 0)
    def _():
        sums_scr[...] = jnp.zeros_like(sums_scr)
        cnts_scr[...] = jnp.zeros_like(cnts_scr)

    dinv = _dinv_from(d0_ref, d1_ref)
    agg = dinv * (a0_ref[...] + a1_ref[...] + hs2_ref[...]) + b2_ref[...]
    out2 = jnp.maximum(agg, 0.0)
    b = batch_ref[...]
    gids = lax.broadcasted_iota(jnp.int32, (RBLK, G), 1)
    onehot = (b == gids).astype(jnp.float32)
    dn = (((0,), (0,)), ((), ()))
    sums_scr[...] += lax.dot_general(onehot, out2, dn,
                                     preferred_element_type=jnp.float32)
    cnts_scr[...] += lax.dot_general(onehot, jnp.ones((RBLK, 1), jnp.float32),
                                     dn, preferred_element_type=jnp.float32)

    @pl.when(i == pl.num_programs(0) - 1)
    def _():
        pooled = sums_scr[...] / jnp.maximum(cnts_scr[...], 1.0)
        res = jnp.dot(pooled, wpool_ref[...],
                      preferred_element_type=jnp.float32)
        out_ref[...] = res + depth_ref[...] * wd_ref[...] + bout_ref[...]


def _row_spec(w):
    return pl.BlockSpec((RBLK, w), lambda i: (i, 0))


def _full_spec(shape):
    return pl.BlockSpec(shape, lambda i: (0, 0))


_GRID = NN // RBLK

_tca_call = pl.pallas_call(
    _tca_body,
    grid=(_GRID,),
    in_specs=[_row_spec(D_IN), _full_spec((D_IN, H)),
              _row_spec(DEGW), _row_spec(DEGW)],
    out_specs=_row_spec(H),
    out_shape=jax.ShapeDtypeStruct((NN, H), jnp.float32),
)

_tcd_call = pl.pallas_call(
    _tcd_body,
    grid=(_GRID,),
    in_specs=[_row_spec(H), _row_spec(H), _row_spec(H),
              _row_spec(DEGW), _row_spec(DEGW),
              _full_spec((1, H)), _full_spec((H, H))],
    out_specs=_row_spec(H),
    out_shape=jax.ShapeDtypeStruct((NN, H), jnp.float32),
)

_tcf_call = pl.pallas_call(
    _tcf_body,
    grid=(_GRID,),
    in_specs=[_row_spec(H), _row_spec(H), _row_spec(H),
              _row_spec(DEGW), _row_spec(DEGW),
              _full_spec((1, H)), _row_spec(1),
              _full_spec((G, 1)), _full_spec((H, 1)),
              _full_spec((1, 1)), _full_spec((1, 1))],
    out_specs=_full_spec((G, 1)),
    out_shape=jax.ShapeDtypeStruct((G, 1), jnp.float32),
    scratch_shapes=[pltpu.VMEM((G, H), jnp.float32),
                    pltpu.VMEM((G, 1), jnp.float32)],
)


def kernel(x, edge_index, batch, depth, W1, b1, W2, b2, W_out, b_out):
    pad = EPAD - E
    trash = jnp.full((pad,), N, jnp.int32)
    srcp = jnp.concatenate([edge_index[0], trash]).reshape(NTILES, NCK, CK)
    dstp = jnp.concatenate([edge_index[1], trash]).reshape(NTILES, NCK, CK)
    xp = jnp.pad(x, ((0, NN - N), (0, 0)))
    batchp = jnp.concatenate(
        [batch, jnp.full((NN - N,), G, jnp.int32)]).reshape(NN, 1)
    ones_rows = jnp.ones((CK, DEGW), jnp.float32)
    zero16 = jnp.zeros((RPT, DEGW), jnp.float32)
    zero64 = jnp.zeros((RPT, H), jnp.float32)
    b1r = b1.reshape(1, H)
    b2r = b2.reshape(1, H)
    depthr = depth.reshape(G, 1)
    wpool = W_out[:H, :]
    wd = W_out[H:, :]
    boutr = b_out.reshape(1, 1)

    degp = _deg_call(dstp, ones_rows, zero16)
    d0, d1 = degp[0], degp[1]
    hs1 = _tca_call(xp, W1, d0, d1)
    a1p = _mp_call(srcp, dstp, hs1, zero64)
    hs2 = _tcd_call(a1p[0], a1p[1], hs1, d0, d1, b1r, W2)
    a2p = _mp_call(srcp, dstp, hs2, zero64)
    res = _tcf_call(a2p[0], a2p[1], hs2, d0, d1, b2r, batchp,
                    depthr, wpool, wd, boutr)
    return res.reshape(G)


# speedup vs baseline: 26.0214x; 1.6370x over previous
import functools

import jax
import jax.numpy as jnp
from jax import lax
from jax.experimental import pallas as pl
from jax.experimental.pallas import tpu as pltpu
from jax.experimental.pallas import tpu_sc as plsc

N = 10000
NN = 10240
E = 320000
H = 64
D_IN = 128
G = 64
NTILES = 32
NSUB = 16
CK = 128
NCK = 84
EPT = E // NTILES
SPT = NN // NTILES
RPT = NN // NSUB
DEGW = 16
RBLK = 2560


def _deg_body(edges_hbm, ones_hbm, zero_hbm, out_hbm, dst_v, ones_v, acc_sh,
              sem):
    c = lax.axis_index("c")
    s = lax.axis_index("s")
    wid = c * NSUB + s
    pltpu.sync_copy(zero_hbm, acc_sh.at[pl.ds(s * RPT, RPT)])
    pltpu.sync_copy(edges_hbm.at[1, wid], dst_v)
    pltpu.sync_copy(ones_hbm, ones_v)
    plsc.subcore_barrier()

    def body(j, carry):
        pltpu.sync_copy(ones_v, acc_sh.at[dst_v.at[j]], add=True)
        return carry

    lax.fori_loop(0, NCK, body, 0)
    plsc.subcore_barrier()
    pltpu.sync_copy(acc_sh.at[pl.ds(s * RPT, RPT)],
                    out_hbm.at[c, pl.ds(s * RPT, RPT)])
    del sem


_deg_call = functools.partial(
    pl.kernel,
    out_type=jax.ShapeDtypeStruct((2, NN, DEGW), jnp.float32),
    mesh=plsc.VectorSubcoreMesh(core_axis_name="c", subcore_axis_name="s"),
    compiler_params=pltpu.CompilerParams(use_tc_tiling_on_sc=False),
    scratch_types=[
        pltpu.VMEM((NCK, CK), jnp.int32),
        pltpu.VMEM((CK, DEGW), jnp.float32),
        pltpu.VMEM_SHARED((NN, DEGW), jnp.float32),
        pltpu.SemaphoreType.DMA,
    ],
)(_deg_body)


NBUF = 4


def _mp_body(edges_hbm, hs_hbm, zero_hbm, out_hbm,
             src_v, dst_v, r0, r1, r2, r3, acc_sh, gsem, ssem):
    c = lax.axis_index("c")
    s = lax.axis_index("s")
    wid = c * NSUB + s
    rbuf = (r0, r1, r2, r3)
    pltpu.sync_copy(zero_hbm, acc_sh.at[pl.ds(s * RPT, RPT)])
    pltpu.sync_copy(edges_hbm.at[0, wid], src_v)
    pltpu.sync_copy(edges_hbm.at[1, wid], dst_v)
    plsc.subcore_barrier()

    for b in range(NBUF):
        pltpu.async_copy(hs_hbm.at[src_v.at[b]], rbuf[b], gsem)

    def grp(i, carry):
        base = i * NBUF
        for b in range(NBUF):
            j = base + b
            pltpu.make_async_copy(hs_hbm.at[src_v.at[j]], rbuf[b],
                                  gsem).wait()
            pltpu.async_copy(rbuf[b], acc_sh.at[dst_v.at[j]], ssem, add=True)
        for b in range(NBUF):
            j = base + b
            pltpu.make_async_copy(rbuf[b], acc_sh.at[dst_v.at[j]],
                                  ssem).wait()

            @pl.when(j + NBUF < NCK)
            def _():
                pltpu.async_copy(hs_hbm.at[src_v.at[j + NBUF]], rbuf[b], gsem)

        return carry

    lax.fori_loop(0, NCK // NBUF, grp, 0)
    plsc.subcore_barrier()
    pltpu.sync_copy(acc_sh.at[pl.ds(s * RPT, RPT)],
                    out_hbm.at[c, pl.ds(s * RPT, RPT), pl.ds(0, H)])


_mp_call = functools.partial(
    pl.kernel,
    out_type=jax.ShapeDtypeStruct((2, NN, 256), jnp.bfloat16),
    mesh=plsc.VectorSubcoreMesh(core_axis_name="c", subcore_axis_name="s"),
    compiler_params=pltpu.CompilerParams(use_tc_tiling_on_sc=False),
    scratch_types=[
        pltpu.VMEM((NCK, CK), jnp.int32),
        pltpu.VMEM((NCK, CK), jnp.int32),
        pltpu.VMEM((CK, H), jnp.bfloat16),
        pltpu.VMEM((CK, H), jnp.bfloat16),
        pltpu.VMEM((CK, H), jnp.bfloat16),
        pltpu.VMEM((CK, H), jnp.bfloat16),
        pltpu.VMEM_SHARED((NN, H), jnp.bfloat16),
        pltpu.SemaphoreType.DMA,
        pltpu.SemaphoreType.DMA,
    ],
)(_mp_body)


def _unpack_pair(ap_ref):
    ai = ap_ref[...][:, :, :H // 2]
    lo = lax.bitcast_convert_type(ai << 16, jnp.float32)
    hi = lax.bitcast_convert_type(ai & jnp.int32(-65536), jnp.float32)
    a = jnp.concatenate([lo, hi], axis=-1)
    return a[0] + a[1]


def _tca_body(x_ref, w1_ref, dp_ref, hs_ref, dinv_ref):
    i = pl.program_id(0)
    d = dp_ref[...]
    dinv = lax.rsqrt(d[0, :, :1] + d[1, :, :1])
    h = jnp.dot(x_ref[...], w1_ref[...], preferred_element_type=jnp.float32)
    rows = i * RBLK + lax.broadcasted_iota(jnp.int32, (RBLK, 1), 0)
    hs_ref[...] = jnp.where(rows < N, h * dinv, 0.0).astype(jnp.bfloat16)
    dinv_ref[...] = jnp.broadcast_to(dinv, (RBLK, H))


def _tcd_body(ap_ref, dinv_ref, b1_ref, w2_ref, hs2_ref):
    dinv = dinv_ref[...]
    agg = dinv * _unpack_pair(ap_ref) + b1_ref[...]
    out1 = jnp.maximum(agg, 0.0)
    h2 = jnp.dot(out1, w2_ref[...], preferred_element_type=jnp.float32)
    hs2_ref[...] = (h2 * dinv).astype(jnp.bfloat16)


def _tcf_body(ap_ref, dinv_ref, b2_ref, batch_ref,
              depth_ref, wpool_ref, wd_ref, bout_ref, out_ref,
              sums_scr, cnts_scr):
    i = pl.program_id(0)

    @pl.when(i == 0)
    def _():
        sums_scr[...] = jnp.zeros_like(sums_scr)
        cnts_scr[...] = jnp.zeros_like(cnts_scr)

    dinv = dinv_ref[...]
    agg = dinv * _unpack_pair(ap_ref) + b2_ref[...]
    out2 = jnp.maximum(agg, 0.0)
    b = batch_ref[...]
    gids = lax.broadcasted_iota(jnp.int32, (RBLK, G), 1)
    onehot = (b == gids).astype(jnp.float32)
    dn = (((0,), (0,)), ((), ()))
    sums_scr[...] += lax.dot_general(onehot, out2, dn,
                                     preferred_element_type=jnp.float32)
    cnts_scr[...] += lax.dot_general(onehot, jnp.ones((RBLK, 1), jnp.float32),
                                     dn, preferred_element_type=jnp.float32)

    @pl.when(i == pl.num_programs(0) - 1)
    def _():
        pooled = sums_scr[...] / jnp.maximum(cnts_scr[...], 1.0)
        res = jnp.dot(pooled, wpool_ref[...],
                      preferred_element_type=jnp.float32)
        out_ref[...] = res + depth_ref[...] * wd_ref[...] + bout_ref[...]


def _row_spec(w):
    return pl.BlockSpec((RBLK, w), lambda i: (i, 0))


def _pair_spec():
    return pl.BlockSpec((2, RBLK, 128), lambda i: (0, i, 0))


def _full_spec(shape):
    return pl.BlockSpec(shape, lambda i: (0, 0))


_GRID = NN // RBLK

_tca_call = pl.pallas_call(
    _tca_body,
    grid=(_GRID,),
    in_specs=[_row_spec(D_IN), _full_spec((D_IN, H)),
              pl.BlockSpec((2, RBLK, DEGW), lambda i: (0, i, 0))],
    out_specs=[_row_spec(H), _row_spec(H)],
    out_shape=[jax.ShapeDtypeStruct((NN, H), jnp.bfloat16),
               jax.ShapeDtypeStruct((NN, H), jnp.float32)],
)

_tcd_call = pl.pallas_call(
    _tcd_body,
    grid=(_GRID,),
    in_specs=[_pair_spec(), _row_spec(H),
              _full_spec((1, H)), _full_spec((H, H))],
    out_specs=_row_spec(H),
    out_shape=jax.ShapeDtypeStruct((NN, H), jnp.bfloat16),
)

_tcf_call = pl.pallas_call(
    _tcf_body,
    grid=(_GRID,),
    in_specs=[_pair_spec(), _row_spec(H),
              _full_spec((1, H)), _row_spec(1),
              _full_spec((G, 1)), _full_spec((H, 1)),
              _full_spec((1, 1)), _full_spec((1, 1))],
    out_specs=_full_spec((G, 1)),
    out_shape=jax.ShapeDtypeStruct((G, 1), jnp.float32),
    scratch_shapes=[pltpu.VMEM((G, H), jnp.float32),
                    pltpu.VMEM((G, 1), jnp.float32)],
)


def _i32_view(a):
    return lax.bitcast_convert_type(a.reshape(2, NN, 128, 2), jnp.int32)


def kernel(x, edge_index, batch, depth, W1, b1, W2, b2, W_out, b_out):
    ppt = NCK * CK - EPT - SPT
    selfs = jnp.arange(NN, dtype=jnp.int32).reshape(NTILES, SPT)
    trash = jnp.broadcast_to(N + jnp.arange(ppt, dtype=jnp.int32) % (NN - N),
                             (NTILES, ppt))
    extra = jnp.concatenate([selfs, trash], axis=1)
    extra2 = jnp.broadcast_to(extra, (2, NTILES, SPT + ppt))
    ep = jnp.concatenate(
        [edge_index.reshape(2, NTILES, EPT), extra2],
        axis=2).reshape(2, NTILES, NCK, CK)
    batchp = jnp.concatenate(
        [batch, jnp.full((NN - N,), G, jnp.int32)]).reshape(NN, 1)
    ones_rows = jnp.ones((CK, DEGW), jnp.float32)
    zero16 = jnp.zeros((RPT, DEGW), jnp.float32)
    zero64 = jnp.zeros((RPT, H), jnp.bfloat16)
    pi = jnp.concatenate([jnp.arange(0, H, 2), jnp.arange(1, H, 2)])
    b1r = b1[pi].reshape(1, H)
    W2p = W2[pi, :]
    b2r = b2[pi].reshape(1, H)
    depthr = depth.reshape(G, 1)
    wpool = W_out[:H, :][pi]
    wd = W_out[H:, :]
    boutr = b_out.reshape(1, 1)

    degp = _deg_call(ep, ones_rows, zero16)
    hs1, dinvp = _tca_call(x, W1, degp)
    a1p = _mp_call(ep, hs1, zero64)
    hs2 = _tcd_call(_i32_view(a1p), dinvp, b1r, W2p)
    a2p = _mp_call(ep, hs2, zero64)
    res = _tcf_call(_i32_view(a2p), dinvp, b2r, batchp,
                    depthr, wpool, wd, boutr)
    return res.reshape(G)


# speedup vs baseline: 45.1867x; 1.7365x over previous
import functools

import jax
import jax.numpy as jnp
from jax import lax
from jax.experimental import pallas as pl
from jax.experimental.pallas import tpu as pltpu
from jax.experimental.pallas import tpu_sc as plsc

N = 10000
NN = 10240
E = 320000
H = 64
D_IN = 128
G = 64
NTILES = 32
NSUB = 16
CK = 128
NCK = 84
EPT = E // NTILES
SPT = NN // NTILES
RPT = NN // NSUB
DEGW = 16
RBLK = 2560


def _deg_body(edges_hbm, ones_hbm, zero_hbm, out_hbm, dst_v, ones_v, acc_sh,
              sem):
    c = lax.axis_index("c")
    s = lax.axis_index("s")
    wid = c * NSUB + s
    pltpu.sync_copy(zero_hbm, acc_sh.at[pl.ds(s * RPT, RPT)])
    pltpu.sync_copy(edges_hbm.at[1, wid], dst_v)
    pltpu.sync_copy(ones_hbm, ones_v)
    plsc.subcore_barrier()

    def body(j, carry):
        pltpu.sync_copy(ones_v, acc_sh.at[dst_v.at[j]], add=True)
        return carry

    lax.fori_loop(0, NCK, body, 0)
    plsc.subcore_barrier()
    pltpu.sync_copy(acc_sh.at[pl.ds(s * RPT, RPT)],
                    out_hbm.at[c, pl.ds(s * RPT, RPT)])
    del sem


_deg_call = functools.partial(
    pl.kernel,
    out_type=jax.ShapeDtypeStruct((2, NN, DEGW), jnp.float32),
    mesh=plsc.VectorSubcoreMesh(core_axis_name="c", subcore_axis_name="s"),
    compiler_params=pltpu.CompilerParams(use_tc_tiling_on_sc=False),
    scratch_types=[
        pltpu.VMEM((NCK, CK), jnp.int32),
        pltpu.VMEM((CK, DEGW), jnp.float32),
        pltpu.VMEM_SHARED((NN, DEGW), jnp.float32),
        pltpu.SemaphoreType.DMA,
    ],
)(_deg_body)


NBUF = 4


def _mp_body(edges_hbm, hs_hbm, zero_hbm, out_hbm,
             src_v, dst_v, r0, r1, r2, r3, acc_sh, gsem, ssem):
    c = lax.axis_index("c")
    s = lax.axis_index("s")
    wid = c * NSUB + s
    rbuf = (r0, r1, r2, r3)
    pltpu.sync_copy(zero_hbm, acc_sh.at[pl.ds(s * RPT, RPT)])
    pltpu.sync_copy(edges_hbm.at[0, wid], src_v)
    pltpu.sync_copy(edges_hbm.at[1, wid], dst_v)
    plsc.subcore_barrier()

    for b in range(NBUF):
        pltpu.async_copy(hs_hbm.at[src_v.at[b]], rbuf[b], gsem)

    def grp(i, carry):
        base = i * NBUF
        for b in range(NBUF):
            j = base + b
            pltpu.make_async_copy(hs_hbm.at[src_v.at[j]], rbuf[b],
                                  gsem).wait()
            pltpu.async_copy(rbuf[b], acc_sh.at[dst_v.at[j]], ssem, add=True)
        for b in range(NBUF):
            j = base + b
            pltpu.make_async_copy(rbuf[b], acc_sh.at[dst_v.at[j]],
                                  ssem).wait()

            @pl.when(j + NBUF < NCK)
            def _():
                pltpu.async_copy(hs_hbm.at[src_v.at[j + NBUF]], rbuf[b], gsem)

        return carry

    lax.fori_loop(0, NCK // NBUF, grp, 0)
    plsc.subcore_barrier()
    pltpu.sync_copy(acc_sh.at[pl.ds(s * RPT, RPT)],
                    out_hbm.at[c, pl.ds(s * RPT, RPT), pl.ds(0, H)])


_mp_call = functools.partial(
    pl.kernel,
    out_type=jax.ShapeDtypeStruct((2, NN, 128), jnp.float32),
    mesh=plsc.VectorSubcoreMesh(core_axis_name="c", subcore_axis_name="s"),
    compiler_params=pltpu.CompilerParams(use_tc_tiling_on_sc=False),
    scratch_types=[
        pltpu.VMEM((NCK, CK), jnp.int32),
        pltpu.VMEM((NCK, CK), jnp.int32),
        pltpu.VMEM((CK, H), jnp.float32),
        pltpu.VMEM((CK, H), jnp.float32),
        pltpu.VMEM((CK, H), jnp.float32),
        pltpu.VMEM((CK, H), jnp.float32),
        pltpu.VMEM_SHARED((NN, H), jnp.float32),
        pltpu.SemaphoreType.DMA,
        pltpu.SemaphoreType.DMA,
    ],
)(_mp_body)


def _unpack_pair(ap_ref):
    a = ap_ref[...][:, :, :H]
    return a[0] + a[1]


def _tca_body(x_ref, w1_ref, dp_ref, hs_ref, dinv_ref):
    i = pl.program_id(0)
    d = dp_ref[...]
    dinv = lax.rsqrt(d[0, :, :1] + d[1, :, :1])
    h = jnp.dot(x_ref[...], w1_ref[...], preferred_element_type=jnp.float32)
    rows = i * RBLK + lax.broadcasted_iota(jnp.int32, (RBLK, 1), 0)
    hs_ref[...] = jnp.where(rows < N, h * dinv, 0.0)
    dinv_ref[...] = jnp.broadcast_to(dinv, (RBLK, H))


def _tcd_body(ap_ref, dinv_ref, b1_ref, w2_ref, hs2_ref):
    dinv = dinv_ref[...]
    agg = dinv * _unpack_pair(ap_ref) + b1_ref[...]
    out1 = jnp.maximum(agg, 0.0)
    h2 = jnp.dot(out1, w2_ref[...], preferred_element_type=jnp.float32)
    hs2_ref[...] = h2 * dinv


def _tcf_body(ap_ref, dinv_ref, b2_ref, batch_ref,
              depth_ref, wpool_ref, wd_ref, bout_ref, out_ref,
              sums_scr, cnts_scr):
    i = pl.program_id(0)

    @pl.when(i == 0)
    def _():
        sums_scr[...] = jnp.zeros_like(sums_scr)
        cnts_scr[...] = jnp.zeros_like(cnts_scr)

    dinv = dinv_ref[...]
    agg = dinv * _unpack_pair(ap_ref) + b2_ref[...]
    out2 = jnp.maximum(agg, 0.0)
    b = batch_ref[...]
    gids = lax.broadcasted_iota(jnp.int32, (RBLK, G), 1)
    onehot = (b == gids).astype(jnp.float32)
    dn = (((0,), (0,)), ((), ()))
    sums_scr[...] += lax.dot_general(onehot, out2, dn,
                                     preferred_element_type=jnp.float32)
    cnts_scr[...] += lax.dot_general(onehot, jnp.ones((RBLK, 1), jnp.float32),
                                     dn, preferred_element_type=jnp.float32)

    @pl.when(i == pl.num_programs(0) - 1)
    def _():
        pooled = sums_scr[...] / jnp.maximum(cnts_scr[...], 1.0)
        res = jnp.dot(pooled, wpool_ref[...],
                      preferred_element_type=jnp.float32)
        out_ref[...] = res + depth_ref[...] * wd_ref[...] + bout_ref[...]


def _row_spec(w):
    return pl.BlockSpec((RBLK, w), lambda i: (i, 0))


def _pair_spec():
    return pl.BlockSpec((2, RBLK, 128), lambda i: (0, i, 0))


def _full_spec(shape):
    return pl.BlockSpec(shape, lambda i: (0, 0))


_GRID = NN // RBLK

_tca_call = pl.pallas_call(
    _tca_body,
    grid=(_GRID,),
    in_specs=[_row_spec(D_IN), _full_spec((D_IN, H)),
              pl.BlockSpec((2, RBLK, DEGW), lambda i: (0, i, 0))],
    out_specs=[_row_spec(H), _row_spec(H)],
    out_shape=[jax.ShapeDtypeStruct((NN, H), jnp.float32),
               jax.ShapeDtypeStruct((NN, H), jnp.float32)],
)

_tcd_call = pl.pallas_call(
    _tcd_body,
    grid=(_GRID,),
    in_specs=[_pair_spec(), _row_spec(H),
              _full_spec((1, H)), _full_spec((H, H))],
    out_specs=_row_spec(H),
    out_shape=jax.ShapeDtypeStruct((NN, H), jnp.float32),
)

_tcf_call = pl.pallas_call(
    _tcf_body,
    grid=(_GRID,),
    in_specs=[_pair_spec(), _row_spec(H),
              _full_spec((1, H)), _row_spec(1),
              _full_spec((G, 1)), _full_spec((H, 1)),
              _full_spec((1, 1)), _full_spec((1, 1))],
    out_specs=_full_spec((G, 1)),
    out_shape=jax.ShapeDtypeStruct((G, 1), jnp.float32),
    scratch_shapes=[pltpu.VMEM((G, H), jnp.float32),
                    pltpu.VMEM((G, 1), jnp.float32)],
)


def kernel(x, edge_index, batch, depth, W1, b1, W2, b2, W_out, b_out):
    ppt = NCK * CK - EPT - SPT
    selfs = jnp.arange(NN, dtype=jnp.int32).reshape(NTILES, SPT)
    trash = jnp.broadcast_to(N + jnp.arange(ppt, dtype=jnp.int32) % (NN - N),
                             (NTILES, ppt))
    extra = jnp.concatenate([selfs, trash], axis=1)
    extra2 = jnp.broadcast_to(extra, (2, NTILES, SPT + ppt))
    ep = jnp.concatenate(
        [edge_index.reshape(2, NTILES, EPT), extra2],
        axis=2).reshape(2, NTILES, NCK, CK)
    batchp = jnp.concatenate(
        [batch, jnp.full((NN - N,), G, jnp.int32)]).reshape(NN, 1)
    ones_rows = jnp.ones((CK, DEGW), jnp.float32)
    zero16 = jnp.zeros((RPT, DEGW), jnp.float32)
    zero64 = jnp.zeros((RPT, H), jnp.float32)
    b1r = b1.reshape(1, H)
    b2r = b2.reshape(1, H)
    depthr = depth.reshape(G, 1)
    wpool = W_out[:H, :]
    wd = W_out[H:, :]
    boutr = b_out.reshape(1, 1)

    degp = _deg_call(ep, ones_rows, zero16)
    hs1, dinvp = _tca_call(x, W1, degp)
    a1p = _mp_call(ep, hs1, zero64)
    hs2 = _tcd_call(a1p, dinvp, b1r, W2)
    a2p = _mp_call(ep, hs2, zero64)
    res = _tcf_call(a2p, dinvp, b2r, batchp,
                    depthr, wpool, wd, boutr)
    return res.reshape(G)


# speedup vs baseline: 46.2208x; 1.0229x over previous
import functools

import jax
import jax.numpy as jnp
from jax import lax
from jax.experimental import pallas as pl
from jax.experimental.pallas import tpu as pltpu
from jax.experimental.pallas import tpu_sc as plsc

N = 10000
NN = 10240
E = 320000
H = 64
D_IN = 128
G = 64
NTILES = 32
NSUB = 16
CK = 128
NCK = 80
EPT = E // NTILES
RPT = NN // NSUB
DEGW = 16
RBLK = 2560


def _deg_body(edges_hbm, ones_hbm, zero_hbm, out_hbm, dst_v, ones_v, acc_sh,
              sem):
    c = lax.axis_index("c")
    s = lax.axis_index("s")
    wid = c * NSUB + s
    pltpu.sync_copy(zero_hbm, acc_sh.at[pl.ds(s * RPT, RPT)])
    pltpu.sync_copy(edges_hbm.at[1, wid], dst_v)
    pltpu.sync_copy(ones_hbm, ones_v)
    plsc.subcore_barrier()

    def body(j, carry):
        pltpu.sync_copy(ones_v, acc_sh.at[dst_v.at[j]], add=True)
        return carry

    lax.fori_loop(0, NCK, body, 0)
    plsc.subcore_barrier()
    pltpu.sync_copy(acc_sh.at[pl.ds(s * RPT, RPT)],
                    out_hbm.at[c, pl.ds(s * RPT, RPT)])
    del sem


_deg_call = functools.partial(
    pl.kernel,
    out_type=jax.ShapeDtypeStruct((2, NN, DEGW), jnp.float32),
    mesh=plsc.VectorSubcoreMesh(core_axis_name="c", subcore_axis_name="s"),
    compiler_params=pltpu.CompilerParams(use_tc_tiling_on_sc=False),
    scratch_types=[
        pltpu.VMEM((NCK, CK), jnp.int32),
        pltpu.VMEM((CK, DEGW), jnp.float32),
        pltpu.VMEM_SHARED((NN, DEGW), jnp.float32),
        pltpu.SemaphoreType.DMA,
    ],
)(_deg_body)


NBUF = 4


def _mp_body(edges_hbm, hs_hbm, zero_hbm, out_hbm,
             src_v, dst_v, r0, r1, r2, r3, acc_sh, gsem, ssem):
    c = lax.axis_index("c")
    s = lax.axis_index("s")
    wid = c * NSUB + s
    rbuf = (r0, r1, r2, r3)
    pltpu.sync_copy(zero_hbm, acc_sh.at[pl.ds(s * RPT, RPT)])
    pltpu.sync_copy(edges_hbm.at[0, wid], src_v)
    pltpu.sync_copy(edges_hbm.at[1, wid], dst_v)
    plsc.subcore_barrier()

    for b in range(NBUF):
        pltpu.async_copy(hs_hbm.at[src_v.at[b]], rbuf[b], gsem)

    def grp(i, carry):
        base = i * NBUF
        for b in range(NBUF):
            j = base + b
            pltpu.make_async_copy(hs_hbm.at[src_v.at[j]], rbuf[b],
                                  gsem).wait()
            pltpu.async_copy(rbuf[b], acc_sh.at[dst_v.at[j]], ssem, add=True)
        for b in range(NBUF):
            j = base + b
            pltpu.make_async_copy(rbuf[b], acc_sh.at[dst_v.at[j]],
                                  ssem).wait()

            @pl.when(j + NBUF < NCK)
            def _():
                pltpu.async_copy(hs_hbm.at[src_v.at[j + NBUF]], rbuf[b], gsem)

        return carry

    lax.fori_loop(0, NCK // NBUF, grp, 0)
    plsc.subcore_barrier()
    pltpu.sync_copy(acc_sh.at[pl.ds(s * RPT, RPT)],
                    out_hbm.at[c, pl.ds(s * RPT, RPT), pl.ds(0, H)])


_mp_call = functools.partial(
    pl.kernel,
    out_type=jax.ShapeDtypeStruct((2, NN, 128), jnp.float32),
    mesh=plsc.VectorSubcoreMesh(core_axis_name="c", subcore_axis_name="s"),
    compiler_params=pltpu.CompilerParams(use_tc_tiling_on_sc=False),
    scratch_types=[
        pltpu.VMEM((NCK, CK), jnp.int32),
        pltpu.VMEM((NCK, CK), jnp.int32),
        pltpu.VMEM((CK, H), jnp.float32),
        pltpu.VMEM((CK, H), jnp.float32),
        pltpu.VMEM((CK, H), jnp.float32),
        pltpu.VMEM((CK, H), jnp.float32),
        pltpu.VMEM_SHARED((NN, H), jnp.float32),
        pltpu.SemaphoreType.DMA,
        pltpu.SemaphoreType.DMA,
    ],
)(_mp_body)


def _merge_pair(ap_ref):
    a = ap_ref[...][:, :, :H]
    return a[0] + a[1]


def _tca_body(x_ref, w1_ref, dp_ref, hs_ref, dinv_ref):
    i = pl.program_id(0)
    d = dp_ref[...]
    dinv = lax.rsqrt(d[0, :, :1] + d[1, :, :1] + 1.0)
    h = jnp.dot(x_ref[...], w1_ref[...], preferred_element_type=jnp.float32)
    rows = i * RBLK + lax.broadcasted_iota(jnp.int32, (RBLK, 1), 0)
    hs_ref[...] = jnp.where(rows < N, h * dinv, 0.0)
    dinv_ref[...] = jnp.broadcast_to(dinv, (RBLK, H))


def _tcd_body(ap_ref, hs1_ref, dinv_ref, b1_ref, w2_ref, hs2_ref):
    dinv = dinv_ref[...]
    agg = dinv * (_merge_pair(ap_ref) + hs1_ref[...]) + b1_ref[...]
    out1 = jnp.maximum(agg, 0.0)
    h2 = jnp.dot(out1, w2_ref[...], preferred_element_type=jnp.float32)
    hs2_ref[...] = h2 * dinv


def _tcf_body(ap_ref, hs2_ref, dinv_ref, b2_ref, batch_ref,
              depth_ref, wpool_ref, wd_ref, bout_ref, out_ref,
              sums_scr, cnts_scr):
    i = pl.program_id(0)

    @pl.when(i == 0)
    def _():
        sums_scr[...] = jnp.zeros_like(sums_scr)
        cnts_scr[...] = jnp.zeros_like(cnts_scr)

    dinv = dinv_ref[...]
    agg = dinv * (_merge_pair(ap_ref) + hs2_ref[...]) + b2_ref[...]
    out2 = jnp.maximum(agg, 0.0)
    b = batch_ref[...]
    gids = lax.broadcasted_iota(jnp.int32, (RBLK, G), 1)
    onehot = (b == gids).astype(jnp.float32)
    dn = (((0,), (0,)), ((), ()))
    sums_scr[...] += lax.dot_general(onehot, out2, dn,
                                     preferred_element_type=jnp.float32)
    cnts_scr[...] += lax.dot_general(onehot, jnp.ones((RBLK, 1), jnp.float32),
                                     dn, preferred_element_type=jnp.float32)

    @pl.when(i == pl.num_programs(0) - 1)
    def _():
        pooled = sums_scr[...] / jnp.maximum(cnts_scr[...], 1.0)
        res = jnp.dot(pooled, wpool_ref[...],
                      preferred_element_type=jnp.float32)
        out_ref[...] = res + depth_ref[...] * wd_ref[...] + bout_ref[...]


def _row_spec(w):
    return pl.BlockSpec((RBLK, w), lambda i: (i, 0))


def _pair_spec():
    return pl.BlockSpec((2, RBLK, 128), lambda i: (0, i, 0))


def _full_spec(shape):
    return pl.BlockSpec(shape, lambda i: (0, 0))


_GRID = NN // RBLK

_tca_call = pl.pallas_call(
    _tca_body,
    grid=(_GRID,),
    in_specs=[_row_spec(D_IN), _full_spec((D_IN, H)),
              pl.BlockSpec((2, RBLK, DEGW), lambda i: (0, i, 0))],
    out_specs=[_row_spec(H), _row_spec(H)],
    out_shape=[jax.ShapeDtypeStruct((NN, H), jnp.float32),
               jax.ShapeDtypeStruct((NN, H), jnp.float32)],
)

_tcd_call = pl.pallas_call(
    _tcd_body,
    grid=(_GRID,),
    in_specs=[_pair_spec(), _row_spec(H), _row_spec(H),
              _full_spec((1, H)), _full_spec((H, H))],
    out_specs=_row_spec(H),
    out_shape=jax.ShapeDtypeStruct((NN, H), jnp.float32),
)

_tcf_call = pl.pallas_call(
    _tcf_body,
    grid=(_GRID,),
    in_specs=[_pair_spec(), _row_spec(H), _row_spec(H),
              _full_spec((1, H)), _row_spec(1),
              _full_spec((G, 1)), _full_spec((H, 1)),
              _full_spec((1, 1)), _full_spec((1, 1))],
    out_specs=_full_spec((G, 1)),
    out_shape=jax.ShapeDtypeStruct((G, 1), jnp.float32),
    scratch_shapes=[pltpu.VMEM((G, H), jnp.float32),
                    pltpu.VMEM((G, 1), jnp.float32)],
)


def kernel(x, edge_index, batch, depth, W1, b1, W2, b2, W_out, b_out):
    ppt = NCK * CK - EPT
    trash = jnp.broadcast_to(N + jnp.arange(ppt, dtype=jnp.int32) % (NN - N),
                             (2, NTILES, ppt))
    ep = jnp.concatenate(
        [edge_index.reshape(2, NTILES, EPT), trash],
        axis=2).reshape(2, NTILES, NCK, CK)
    batchp = jnp.concatenate(
        [batch, jnp.full((NN - N,), G, jnp.int32)]).reshape(NN, 1)
    ones_rows = jnp.ones((CK, DEGW), jnp.float32)
    zero16 = jnp.zeros((RPT, DEGW), jnp.float32)
    zero64 = jnp.zeros((RPT, H), jnp.float32)
    b1r = b1.reshape(1, H)
    b2r = b2.reshape(1, H)
    depthr = depth.reshape(G, 1)
    wpool = W_out[:H, :]
    wd = W_out[H:, :]
    boutr = b_out.reshape(1, 1)

    degp = _deg_call(ep, ones_rows, zero16)
    hs1, dinvp = _tca_call(x, W1, degp)
    a1p = _mp_call(ep, hs1, zero64)
    hs2 = _tcd_call(a1p, hs1, dinvp, b1r, W2)
    a2p = _mp_call(ep, hs2, zero64)
    res = _tcf_call(a2p, hs2, dinvp, b2r, batchp,
                    depthr, wpool, wd, boutr)
    return res.reshape(G)


# speedup vs baseline: 47.2975x; 1.0233x over previous
import functools

import jax
import jax.numpy as jnp
from jax import lax
from jax.experimental import pallas as pl
from jax.experimental.pallas import tpu as pltpu
from jax.experimental.pallas import tpu_sc as plsc

N = 10000
NN = 10240
E = 320000
H = 64
D_IN = 128
G = 64
NTILES = 32
NSUB = 16
CK = 128
NCK = 80
EPT = E // NTILES
RPT = NN // NSUB
DEGW = 16
RBLK = 2560


def _deg_body(edges_hbm, ones_hbm, zero_hbm, out_hbm, dst_v, ones_v, acc_sh,
              sem):
    c = lax.axis_index("c")
    s = lax.axis_index("s")
    wid = c * NSUB + s
    pltpu.sync_copy(zero_hbm, acc_sh.at[pl.ds(s * RPT, RPT)])
    pltpu.sync_copy(edges_hbm.at[1, wid], dst_v)
    pltpu.sync_copy(ones_hbm, ones_v)
    plsc.subcore_barrier()

    def body(j, carry):
        pltpu.sync_copy(ones_v, acc_sh.at[dst_v.at[j]], add=True)
        return carry

    lax.fori_loop(0, NCK, body, 0)
    plsc.subcore_barrier()
    pltpu.sync_copy(acc_sh.at[pl.ds(s * RPT, RPT)],
                    out_hbm.at[c, pl.ds(s * RPT, RPT)])
    del sem


_deg_call = functools.partial(
    pl.kernel,
    out_type=jax.ShapeDtypeStruct((2, NN, DEGW), jnp.float32),
    mesh=plsc.VectorSubcoreMesh(core_axis_name="c", subcore_axis_name="s"),
    compiler_params=pltpu.CompilerParams(use_tc_tiling_on_sc=False),
    scratch_types=[
        pltpu.VMEM((NCK, CK), jnp.int32),
        pltpu.VMEM((CK, DEGW), jnp.float32),
        pltpu.VMEM_SHARED((NN, DEGW), jnp.float32),
        pltpu.SemaphoreType.DMA,
    ],
)(_deg_body)


NBUF = 8


def _mp_body(edges_hbm, hs_hbm, zero_hbm, out_hbm,
             src_v, dst_v, r0, r1, r2, r3, r4, r5, r6, r7, acc_sh,
             gsem, ssem):
    c = lax.axis_index("c")
    s = lax.axis_index("s")
    wid = c * NSUB + s
    rbuf = (r0, r1, r2, r3, r4, r5, r6, r7)
    pltpu.sync_copy(zero_hbm, acc_sh.at[pl.ds(s * RPT, RPT)])
    pltpu.sync_copy(edges_hbm.at[0, wid], src_v)
    pltpu.sync_copy(edges_hbm.at[1, wid], dst_v)
    plsc.subcore_barrier()

    for b in range(NBUF):
        pltpu.async_copy(hs_hbm.at[src_v.at[b]], rbuf[b], gsem)

    def grp(i, carry):
        base = i * NBUF
        for b in range(NBUF):
            j = base + b
            pltpu.make_async_copy(hs_hbm.at[src_v.at[j]], rbuf[b],
                                  gsem).wait()
            pltpu.async_copy(rbuf[b], acc_sh.at[dst_v.at[j]], ssem, add=True)
        for b in range(NBUF):
            j = base + b
            pltpu.make_async_copy(rbuf[b], acc_sh.at[dst_v.at[j]],
                                  ssem).wait()

            @pl.when(j + NBUF < NCK)
            def _():
                pltpu.async_copy(hs_hbm.at[src_v.at[j + NBUF]], rbuf[b], gsem)

        return carry

    lax.fori_loop(0, NCK // NBUF, grp, 0)
    plsc.subcore_barrier()
    pltpu.sync_copy(acc_sh.at[pl.ds(s * RPT, RPT)],
                    out_hbm.at[c, pl.ds(s * RPT, RPT), pl.ds(0, H)])


_mp_call = functools.partial(
    pl.kernel,
    out_type=jax.ShapeDtypeStruct((2, NN, 128), jnp.float32),
    mesh=plsc.VectorSubcoreMesh(core_axis_name="c", subcore_axis_name="s"),
    compiler_params=pltpu.CompilerParams(use_tc_tiling_on_sc=False),
    scratch_types=[
        pltpu.VMEM((NCK, CK), jnp.int32),
        pltpu.VMEM((NCK, CK), jnp.int32),
        pltpu.VMEM((CK, H), jnp.float32),
        pltpu.VMEM((CK, H), jnp.float32),
        pltpu.VMEM((CK, H), jnp.float32),
        pltpu.VMEM((CK, H), jnp.float32),
        pltpu.VMEM((CK, H), jnp.float32),
        pltpu.VMEM((CK, H), jnp.float32),
        pltpu.VMEM((CK, H), jnp.float32),
        pltpu.VMEM((CK, H), jnp.float32),
        pltpu.VMEM_SHARED((NN, H), jnp.float32),
        pltpu.SemaphoreType.DMA,
        pltpu.SemaphoreType.DMA,
    ],
)(_mp_body)


def _merge_pair(ap_ref):
    a = ap_ref[...][:, :, :H]
    return a[0] + a[1]


def _tca_body(x_ref, w1_ref, dp_ref, hs_ref, dinv_ref):
    i = pl.program_id(0)
    d = dp_ref[...]
    dinv = lax.rsqrt(d[0, :, :1] + d[1, :, :1] + 1.0)
    h = jnp.dot(x_ref[...], w1_ref[...], preferred_element_type=jnp.float32)
    rows = i * RBLK + lax.broadcasted_iota(jnp.int32, (RBLK, 1), 0)
    hs_ref[...] = jnp.where(rows < N, h * dinv, 0.0)
    dinv_ref[...] = jnp.broadcast_to(dinv, (RBLK, H))


def _tcd_body(ap_ref, hs1_ref, dinv_ref, b1_ref, w2_ref, hs2_ref):
    dinv = dinv_ref[...]
    agg = dinv * (_merge_pair(ap_ref) + hs1_ref[...]) + b1_ref[...]
    out1 = jnp.maximum(agg, 0.0)
    h2 = jnp.dot(out1, w2_ref[...], preferred_element_type=jnp.float32)
    hs2_ref[...] = h2 * dinv


def _tcf_body(ap_ref, hs2_ref, dinv_ref, b2_ref, batch_ref,
              depth_ref, wpool_ref, wd_ref, bout_ref, out_ref,
              sums_scr, cnts_scr):
    i = pl.program_id(0)

    @pl.when(i == 0)
    def _():
        sums_scr[...] = jnp.zeros_like(sums_scr)
        cnts_scr[...] = jnp.zeros_like(cnts_scr)

    dinv = dinv_ref[...]
    agg = dinv * (_merge_pair(ap_ref) + hs2_ref[...]) + b2_ref[...]
    out2 = jnp.maximum(agg, 0.0)
    b = batch_ref[...]
    gids = lax.broadcasted_iota(jnp.int32, (RBLK, G), 1)
    onehot = (b == gids).astype(jnp.float32)
    dn = (((0,), (0,)), ((), ()))
    sums_scr[...] += lax.dot_general(onehot, out2, dn,
                                     preferred_element_type=jnp.float32)
    cnts_scr[...] += lax.dot_general(onehot, jnp.ones((RBLK, 1), jnp.float32),
                                     dn, preferred_element_type=jnp.float32)

    @pl.when(i == pl.num_programs(0) - 1)
    def _():
        pooled = sums_scr[...] / jnp.maximum(cnts_scr[...], 1.0)
        res = jnp.dot(pooled, wpool_ref[...],
                      preferred_element_type=jnp.float32)
        out_ref[...] = res + depth_ref[...] * wd_ref[...] + bout_ref[...]


def _row_spec(w):
    return pl.BlockSpec((RBLK, w), lambda i: (i, 0))


def _pair_spec():
    return pl.BlockSpec((2, RBLK, 128), lambda i: (0, i, 0))


def _full_spec(shape):
    return pl.BlockSpec(shape, lambda i: (0, 0))


_GRID = NN // RBLK

_tca_call = pl.pallas_call(
    _tca_body,
    grid=(_GRID,),
    in_specs=[_row_spec(D_IN), _full_spec((D_IN, H)),
              pl.BlockSpec((2, RBLK, DEGW), lambda i: (0, i, 0))],
    out_specs=[_row_spec(H), _row_spec(H)],
    out_shape=[jax.ShapeDtypeStruct((NN, H), jnp.float32),
               jax.ShapeDtypeStruct((NN, H), jnp.float32)],
)

_tcd_call = pl.pallas_call(
    _tcd_body,
    grid=(_GRID,),
    in_specs=[_pair_spec(), _row_spec(H), _row_spec(H),
              _full_spec((1, H)), _full_spec((H, H))],
    out_specs=_row_spec(H),
    out_shape=jax.ShapeDtypeStruct((NN, H), jnp.float32),
)

_tcf_call = pl.pallas_call(
    _tcf_body,
    grid=(_GRID,),
    in_specs=[_pair_spec(), _row_spec(H), _row_spec(H),
              _full_spec((1, H)), _row_spec(1),
              _full_spec((G, 1)), _full_spec((H, 1)),
              _full_spec((1, 1)), _full_spec((1, 1))],
    out_specs=_full_spec((G, 1)),
    out_shape=jax.ShapeDtypeStruct((G, 1), jnp.float32),
    scratch_shapes=[pltpu.VMEM((G, H), jnp.float32),
                    pltpu.VMEM((G, 1), jnp.float32)],
)


def kernel(x, edge_index, batch, depth, W1, b1, W2, b2, W_out, b_out):
    ppt = NCK * CK - EPT
    trash = jnp.broadcast_to(N + jnp.arange(ppt, dtype=jnp.int32) % (NN - N),
                             (2, NTILES, ppt))
    ep = jnp.concatenate(
        [edge_index.reshape(2, NTILES, EPT), trash],
        axis=2).reshape(2, NTILES, NCK, CK)
    batchp = jnp.concatenate(
        [batch, jnp.full((NN - N,), G, jnp.int32)]).reshape(NN, 1)
    ones_rows = jnp.ones((CK, DEGW), jnp.float32)
    zero16 = jnp.zeros((RPT, DEGW), jnp.float32)
    zero64 = jnp.zeros((RPT, H), jnp.float32)
    b1r = b1.reshape(1, H)
    b2r = b2.reshape(1, H)
    depthr = depth.reshape(G, 1)
    wpool = W_out[:H, :]
    wd = W_out[H:, :]
    boutr = b_out.reshape(1, 1)

    degp = _deg_call(ep, ones_rows, zero16)
    hs1, dinvp = _tca_call(x, W1, degp)
    a1p = _mp_call(ep, hs1, zero64)
    hs2 = _tcd_call(a1p, hs1, dinvp, b1r, W2)
    a2p = _mp_call(ep, hs2, zero64)
    res = _tcf_call(a2p, hs2, dinvp, b2r, batchp,
                    depthr, wpool, wd, boutr)
    return res.reshape(G)


# speedup vs baseline: 48.5674x; 1.0269x over previous
import functools

import jax
import jax.numpy as jnp
from jax import lax
from jax.experimental import pallas as pl
from jax.experimental.pallas import tpu as pltpu
from jax.experimental.pallas import tpu_sc as plsc

N = 10000
NN = 10240
E = 320000
H = 64
D_IN = 128
G = 64
NTILES = 32
NSUB = 16
CK = 128
NCK = 80
EPT = E // NTILES
RPT = NN // NSUB
DEGW = 16
RBLK = 2560


def _deg_body(edges_hbm, ones_hbm, zero_hbm, out_hbm, dst_v, ones_v, acc_sh,
              sem):
    c = lax.axis_index("c")
    s = lax.axis_index("s")
    wid = c * NSUB + s
    pltpu.sync_copy(zero_hbm, acc_sh.at[pl.ds(s * RPT, RPT)])
    pltpu.sync_copy(edges_hbm.at[1, wid], dst_v)
    pltpu.sync_copy(ones_hbm, ones_v)
    plsc.subcore_barrier()

    def body(j, carry):
        pltpu.sync_copy(ones_v, acc_sh.at[dst_v.at[j]], add=True)
        return carry

    lax.fori_loop(0, NCK, body, 0)
    plsc.subcore_barrier()
    pltpu.sync_copy(acc_sh.at[pl.ds(s * RPT, RPT)],
                    out_hbm.at[c, pl.ds(s * RPT, RPT), pl.ds(0, DEGW)])
    del sem


_deg_call = functools.partial(
    pl.kernel,
    out_type=jax.ShapeDtypeStruct((2, NN, 128), jnp.float32),
    mesh=plsc.VectorSubcoreMesh(core_axis_name="c", subcore_axis_name="s"),
    compiler_params=pltpu.CompilerParams(use_tc_tiling_on_sc=False),
    scratch_types=[
        pltpu.VMEM((NCK, CK), jnp.int32),
        pltpu.VMEM((CK, DEGW), jnp.float32),
        pltpu.VMEM_SHARED((NN, DEGW), jnp.float32),
        pltpu.SemaphoreType.DMA,
    ],
)(_deg_body)


NBUF = 8


def _mp_body(edges_hbm, hs_hbm, zero_hbm, out_hbm,
             src_v, dst_v, r0, r1, r2, r3, r4, r5, r6, r7, acc_sh,
             gsem, ssem):
    c = lax.axis_index("c")
    s = lax.axis_index("s")
    wid = c * NSUB + s
    rbuf = (r0, r1, r2, r3, r4, r5, r6, r7)
    pltpu.sync_copy(zero_hbm, acc_sh.at[pl.ds(s * RPT, RPT)])
    pltpu.sync_copy(edges_hbm.at[0, wid], src_v)
    pltpu.sync_copy(edges_hbm.at[1, wid], dst_v)
    plsc.subcore_barrier()

    for b in range(NBUF):
        pltpu.async_copy(hs_hbm.at[src_v.at[b]], rbuf[b], gsem)

    def grp(i, carry):
        base = i * NBUF
        for b in range(NBUF):
            j = base + b
            pltpu.make_async_copy(hs_hbm.at[src_v.at[j]], rbuf[b],
                                  gsem).wait()
            pltpu.async_copy(rbuf[b], acc_sh.at[dst_v.at[j]], ssem, add=True)
        for b in range(NBUF):
            j = base + b
            pltpu.make_async_copy(rbuf[b], acc_sh.at[dst_v.at[j]],
                                  ssem).wait()

            @pl.when(j + NBUF < NCK)
            def _():
                pltpu.async_copy(hs_hbm.at[src_v.at[j + NBUF]], rbuf[b], gsem)

        return carry

    lax.fori_loop(0, NCK // NBUF, grp, 0)
    plsc.subcore_barrier()
    pltpu.sync_copy(acc_sh.at[pl.ds(s * RPT, RPT)],
                    out_hbm.at[c, pl.ds(s * RPT, RPT), pl.ds(0, H)])


_mp_call = functools.partial(
    pl.kernel,
    out_type=jax.ShapeDtypeStruct((2, NN, 128), jnp.float32),
    mesh=plsc.VectorSubcoreMesh(core_axis_name="c", subcore_axis_name="s"),
    compiler_params=pltpu.CompilerParams(use_tc_tiling_on_sc=False),
    scratch_types=[
        pltpu.VMEM((NCK, CK), jnp.int32),
        pltpu.VMEM((NCK, CK), jnp.int32),
        pltpu.VMEM((CK, H), jnp.float32),
        pltpu.VMEM((CK, H), jnp.float32),
        pltpu.VMEM((CK, H), jnp.float32),
        pltpu.VMEM((CK, H), jnp.float32),
        pltpu.VMEM((CK, H), jnp.float32),
        pltpu.VMEM((CK, H), jnp.float32),
        pltpu.VMEM((CK, H), jnp.float32),
        pltpu.VMEM((CK, H), jnp.float32),
        pltpu.VMEM_SHARED((NN, H), jnp.float32),
        pltpu.SemaphoreType.DMA,
        pltpu.SemaphoreType.DMA,
    ],
)(_mp_body)


def _merge_pair(ap_ref):
    a = ap_ref[...][:, :, :H]
    return a[0] + a[1]


def _tca_body(x_ref, w1_ref, dp_ref, hs_ref, dinv_ref):
    i = pl.program_id(0)
    d = dp_ref[...]
    dinv = lax.rsqrt(d[0, :, :1] + d[1, :, :1] + 1.0)
    h = jnp.dot(x_ref[...], w1_ref[...], preferred_element_type=jnp.float32)
    rows = i * RBLK + lax.broadcasted_iota(jnp.int32, (RBLK, 1), 0)
    hs_ref[...] = jnp.where(rows < N, h * dinv, 0.0)
    dinv_ref[...] = jnp.broadcast_to(dinv, (RBLK, H))


def _tcd_body(ap_ref, hs1_ref, dinv_ref, b1_ref, w2_ref, hs2_ref):
    dinv = dinv_ref[...]
    agg = dinv * (_merge_pair(ap_ref) + hs1_ref[...]) + b1_ref[...]
    out1 = jnp.maximum(agg, 0.0)
    h2 = jnp.dot(out1, w2_ref[...], preferred_element_type=jnp.float32)
    hs2_ref[...] = h2 * dinv


def _tcf_body(ap_ref, hs2_ref, dinv_ref, b2_ref, batch_ref,
              depth_ref, wpool_ref, wd_ref, bout_ref, out_ref,
              sums_scr, cnts_scr):
    i = pl.program_id(0)

    @pl.when(i == 0)
    def _():
        sums_scr[...] = jnp.zeros_like(sums_scr)
        cnts_scr[...] = jnp.zeros_like(cnts_scr)

    dinv = dinv_ref[...]
    agg = dinv * (_merge_pair(ap_ref) + hs2_ref[...]) + b2_ref[...]
    out2 = jnp.maximum(agg, 0.0)
    b = batch_ref[...]
    gids = lax.broadcasted_iota(jnp.int32, (RBLK, G), 1)
    onehot = (b == gids).astype(jnp.float32)
    dn = (((0,), (0,)), ((), ()))
    sums_scr[...] += lax.dot_general(onehot, out2, dn,
                                     preferred_element_type=jnp.float32)
    cnts_scr[...] += lax.dot_general(onehot, jnp.ones((RBLK, 1), jnp.float32),
                                     dn, preferred_element_type=jnp.float32)

    @pl.when(i == pl.num_programs(0) - 1)
    def _():
        pooled = sums_scr[...] / jnp.maximum(cnts_scr[...], 1.0)
        res = jnp.dot(pooled, wpool_ref[...],
                      preferred_element_type=jnp.float32)
        out_ref[...] = res + depth_ref[...] * wd_ref[...] + bout_ref[...]


def _row_spec(w):
    return pl.BlockSpec((RBLK, w), lambda i: (i, 0))


def _pair_spec():
    return pl.BlockSpec((2, RBLK, 128), lambda i: (0, i, 0))


def _full_spec(shape):
    return pl.BlockSpec(shape, lambda i: (0, 0))


_GRID = NN // RBLK

_tca_call = pl.pallas_call(
    _tca_body,
    grid=(_GRID,),
    in_specs=[_row_spec(D_IN), _full_spec((D_IN, H)),
              _pair_spec()],
    out_specs=[_row_spec(H), _row_spec(H)],
    out_shape=[jax.ShapeDtypeStruct((NN, H), jnp.float32),
               jax.ShapeDtypeStruct((NN, H), jnp.float32)],
)

_tcd_call = pl.pallas_call(
    _tcd_body,
    grid=(_GRID,),
    in_specs=[_pair_spec(), _row_spec(H), _row_spec(H),
              _full_spec((1, H)), _full_spec((H, H))],
    out_specs=_row_spec(H),
    out_shape=jax.ShapeDtypeStruct((NN, H), jnp.float32),
)

_tcf_call = pl.pallas_call(
    _tcf_body,
    grid=(_GRID,),
    in_specs=[_pair_spec(), _row_spec(H), _row_spec(H),
              _full_spec((1, H)), _row_spec(1),
              _full_spec((G, 1)), _full_spec((H, 1)),
              _full_spec((1, 1)), _full_spec((1, 1))],
    out_specs=_full_spec((G, 1)),
    out_shape=jax.ShapeDtypeStruct((G, 1), jnp.float32),
    scratch_shapes=[pltpu.VMEM((G, H), jnp.float32),
                    pltpu.VMEM((G, 1), jnp.float32)],
)


def kernel(x, edge_index, batch, depth, W1, b1, W2, b2, W_out, b_out):
    ppt = NCK * CK - EPT
    trash = jnp.broadcast_to(N + jnp.arange(ppt, dtype=jnp.int32) % (NN - N),
                             (2, NTILES, ppt))
    ep = jnp.concatenate(
        [edge_index.reshape(2, NTILES, EPT), trash],
        axis=2).reshape(2, NTILES, NCK, CK)
    batchp = jnp.concatenate(
        [batch, jnp.full((NN - N,), G, jnp.int32)]).reshape(NN, 1)
    ones_rows = jnp.ones((CK, DEGW), jnp.float32)
    zero16 = jnp.zeros((RPT, DEGW), jnp.float32)
    zero64 = jnp.zeros((RPT, H), jnp.float32)
    b1r = b1.reshape(1, H)
    b2r = b2.reshape(1, H)
    depthr = depth.reshape(G, 1)
    wpool = W_out[:H, :]
    wd = W_out[H:, :]
    boutr = b_out.reshape(1, 1)

    degp = _deg_call(ep, ones_rows, zero16)
    hs1, dinvp = _tca_call(x, W1, degp)
    a1p = _mp_call(ep, hs1, zero64)
    hs2 = _tcd_call(a1p, hs1, dinvp, b1r, W2)
    a2p = _mp_call(ep, hs2, zero64)
    res = _tcf_call(a2p, hs2, dinvp, b2r, batchp,
                    depthr, wpool, wd, boutr)
    return res.reshape(G)
